# Initial kernel scaffold; baseline (speedup 1.0000x reference)
import jax
import jax.numpy as jnp
from jax.experimental import pallas as pl


def kernel(x, edge_index, Wl1, Wr1, b1, Wl2, Wr2, b2):
    raise NotImplementedError("write your pallas kernel here")


# speedup vs baseline: 2.6379x; 2.6379x over previous
import functools

import jax
import jax.numpy as jnp
from jax import lax
from jax.experimental import pallas as pl
from jax.experimental.pallas import tpu as pltpu
from jax.experimental.pallas import tpu_sc as plsc

_N = 10000
_E = 320000
_D = 128

_NC = 1
_NS = 16
_NW = _NC * _NS

_NP = 10240
_EPT = _E // _NW
_CH = 80
_NCHUNK = _EPT // _CH
_RPT = _NP // _NS
_RB = 1024


def _zero_fill(ref, val):
    v = jnp.full((16,), val, jnp.float32)

    def body(r, _):
        for j in range(_D // 16):
            ref[r, pl.ds(j * 16, 16)] = v
        return 0

    lax.fori_loop(0, _CH, body, 0)


def _count_sc(edge_flat):
    mesh = plsc.VectorSubcoreMesh(
        core_axis_name="c", subcore_axis_name="s", num_cores=_NC)

    @functools.partial(
        pl.kernel,
        mesh=mesh,
        out_type=jax.ShapeDtypeStruct((_NC, _NP, _D), jnp.float32),
        scratch_types=[
            pltpu.VMEM((_CH,), jnp.int32),
            pltpu.VMEM((_CH, _D), jnp.float32),
            pltpu.VMEM((_CH, _D), jnp.float32),
            pltpu.VMEM_SHARED((_NP, _D), jnp.float32),
        ],
    )
    def k(edge_hbm, cnt_out, idx_dst, ones_b, zeros_b, cnt_sh):
        cid = lax.axis_index("c")
        sid = lax.axis_index("s")
        _zero_fill(ones_b, 1.0)
        _zero_fill(zeros_b, 0.0)
        for j in range(_RPT // _CH):
            pltpu.sync_copy(
                zeros_b, cnt_sh.at[pl.ds(sid * _RPT + j * _CH, _CH)])
        plsc.subcore_barrier()

        ebase = (cid * _NS + sid) * _EPT

        def body(c, _):
            off = pl.multiple_of(ebase + c * _CH, 8)
            pltpu.sync_copy(edge_hbm.at[pl.ds(_E + off, _CH)], idx_dst)
            pltpu.sync_copy(ones_b, cnt_sh.at[idx_dst], add=True)
            return 0

        lax.fori_loop(0, _NCHUNK, body, 0)
        plsc.subcore_barrier()

        obase = pl.multiple_of(sid * _RPT, 8)
        pltpu.sync_copy(cnt_sh.at[pl.ds(obase, _RPT)],
                        cnt_out.at[cid, pl.ds(obase, _RPT)])

    return k(edge_flat)


def _seg_sum_sc(feat, edge_flat):
    mesh = plsc.VectorSubcoreMesh(
        core_axis_name="c", subcore_axis_name="s", num_cores=_NC)

    @functools.partial(
        pl.kernel,
        mesh=mesh,
        out_type=jax.ShapeDtypeStruct((_NC, _NP, _D), jnp.float32),
        scratch_types=[
            pltpu.VMEM((_CH,), jnp.int32),
            pltpu.VMEM((_CH,), jnp.int32),
            pltpu.VMEM((_CH, _D), jnp.float32),
            pltpu.VMEM_SHARED((_NP, _D), jnp.float32),
            pltpu.SemaphoreType.DMA,
        ],
    )
    def k(feat_hbm, edge_hbm, acc_out, idx_src, idx_dst, rows, acc_sh, sem):
        cid = lax.axis_index("c")
        sid = lax.axis_index("s")
        _zero_fill(rows, 0.0)
        for j in range(_RPT // _CH):
            pltpu.sync_copy(
                rows, acc_sh.at[pl.ds(sid * _RPT + j * _CH, _CH)])
        plsc.subcore_barrier()

        ebase = (cid * _NS + sid) * _EPT

        def body(c, _):
            off = pl.multiple_of(ebase + c * _CH, 8)
            pltpu.sync_copy(edge_hbm.at[pl.ds(off, _CH)], idx_src)
            pltpu.sync_copy(edge_hbm.at[pl.ds(_E + off, _CH)], idx_dst)
            pltpu.async_copy(feat_hbm.at[idx_src], rows, sem).wait()
            pltpu.sync_copy(rows, acc_sh.at[idx_dst], add=True)
            return 0

        lax.fori_loop(0, _NCHUNK, body, 0)
        plsc.subcore_barrier()

        obase = pl.multiple_of(sid * _RPT, 8)
        pltpu.sync_copy(acc_sh.at[pl.ds(obase, _RPT)],
                        acc_out.at[cid, pl.ds(obase, _RPT)])

    return k(feat, edge_flat)


def _dense_tc(partials, counts, feat, Wl, Wr, b, relu):
    def body(p_ref, c_ref, f_ref, wl_ref, wr_ref, b_ref, o_ref):
        cnt = jnp.sum(c_ref[...], axis=0)
        recip = 1.0 / jnp.maximum(cnt, 1.0)
        agg = jnp.sum(p_ref[...], axis=0) * recip
        y = (jnp.dot(agg, wl_ref[...], preferred_element_type=jnp.float32)
             + jnp.dot(f_ref[...], wr_ref[...],
                       preferred_element_type=jnp.float32)
             + b_ref[...])
        if relu:
            y = jnp.maximum(y, 0.0)
        o_ref[...] = y

    return pl.pallas_call(
        body,
        grid=(_NP // _RB,),
        in_specs=[
            pl.BlockSpec((_NC, _RB, _D), lambda i: (0, i, 0)),
            pl.BlockSpec((_NC, _RB, 1), lambda i: (0, i, 0)),
            pl.BlockSpec((_RB, _D), lambda i: (i, 0)),
            pl.BlockSpec((_D, _D), lambda i: (0, 0)),
            pl.BlockSpec((_D, _D), lambda i: (0, 0)),
            pl.BlockSpec((1, _D), lambda i: (0, 0)),
        ],
        out_specs=pl.BlockSpec((_RB, _D), lambda i: (i, 0)),
        out_shape=jax.ShapeDtypeStruct((_N, _D), jnp.float32),
    )(partials, counts, feat, Wl, Wr, b)


def kernel(x, edge_index, Wl1, Wr1, b1, Wl2, Wr2, b2):
    eflat = edge_index.reshape(-1)
    cnt = _count_sc(eflat)[:, :, :1]
    p1 = _seg_sum_sc(x, eflat)
    h = _dense_tc(p1, cnt, x, Wl1, Wr1, b1.reshape(1, _D), relu=True)
    p2 = _seg_sum_sc(h, eflat)
    out = _dense_tc(p2, cnt, h, Wl2, Wr2, b2.reshape(1, _D), relu=False)
    return out


# speedup vs baseline: 4.9255x; 1.8672x over previous
import functools

import jax
import jax.numpy as jnp
from jax import lax
from jax.experimental import pallas as pl
from jax.experimental.pallas import tpu as pltpu
from jax.experimental.pallas import tpu_sc as plsc

_N = 10000
_E = 320000
_D = 128

_NC = 2
_NS = 16
_NW = _NC * _NS

_NP = 10240
_EPT = _E // _NW
_CH = 80
_NCHUNK = _EPT // _CH
_RPT = _NP // _NS
_RB = 1024


def _zero_fill(ref, val):
    v = jnp.full((16,), val, jnp.float32)

    def body(r, _):
        for j in range(_D // 16):
            ref[r, pl.ds(j * 16, 16)] = v
        return 0

    lax.fori_loop(0, _CH, body, 0)


def _count_sc(edge_flat):
    mesh = plsc.VectorSubcoreMesh(
        core_axis_name="c", subcore_axis_name="s", num_cores=_NC)

    @functools.partial(
        pl.kernel,
        mesh=mesh,
        out_type=jax.ShapeDtypeStruct((_NC, _NP, _D), jnp.float32),
        scratch_types=[
            pltpu.VMEM((_CH,), jnp.int32),
            pltpu.VMEM((_CH, _D), jnp.float32),
            pltpu.VMEM((_CH, _D), jnp.float32),
            pltpu.VMEM_SHARED((_NP, _D), jnp.float32),
        ],
    )
    def k(edge_hbm, cnt_out, idx_dst, ones_b, zeros_b, cnt_sh):
        cid = lax.axis_index("c")
        sid = lax.axis_index("s")
        _zero_fill(ones_b, 1.0)
        _zero_fill(zeros_b, 0.0)
        for j in range(_RPT // _CH):
            pltpu.sync_copy(
                zeros_b, cnt_sh.at[pl.ds(sid * _RPT + j * _CH, _CH)])
        plsc.subcore_barrier()

        ebase = (cid * _NS + sid) * _EPT

        def body(c, _):
            off = pl.multiple_of(ebase + c * _CH, 8)
            pltpu.sync_copy(edge_hbm.at[pl.ds(_E + off, _CH)], idx_dst)
            pltpu.sync_copy(ones_b, cnt_sh.at[idx_dst], add=True)
            return 0

        lax.fori_loop(0, _NCHUNK, body, 0)
        plsc.subcore_barrier()

        obase = pl.multiple_of(sid * _RPT, 8)
        pltpu.sync_copy(cnt_sh.at[pl.ds(obase, _RPT)],
                        cnt_out.at[cid, pl.ds(obase, _RPT)])

    return k(edge_flat)


def _seg_sum_sc(feat, edge_flat):
    mesh = plsc.VectorSubcoreMesh(
        core_axis_name="c", subcore_axis_name="s", num_cores=_NC)

    @functools.partial(
        pl.kernel,
        mesh=mesh,
        out_type=jax.ShapeDtypeStruct((_NC, _NP, _D), jnp.float32),
        scratch_types=[
            pltpu.VMEM((_CH,), jnp.int32),
            pltpu.VMEM((_CH,), jnp.int32),
            pltpu.VMEM((_CH, _D), jnp.float32),
            pltpu.VMEM_SHARED((_NP, _D), jnp.float32),
            pltpu.SemaphoreType.DMA,
        ],
    )
    def k(feat_hbm, edge_hbm, acc_out, idx_src, idx_dst, rows, acc_sh, sem):
        cid = lax.axis_index("c")
        sid = lax.axis_index("s")
        _zero_fill(rows, 0.0)
        for j in range(_RPT // _CH):
            pltpu.sync_copy(
                rows, acc_sh.at[pl.ds(sid * _RPT + j * _CH, _CH)])
        plsc.subcore_barrier()

        ebase = (cid * _NS + sid) * _EPT

        def body(c, _):
            off = pl.multiple_of(ebase + c * _CH, 8)
            pltpu.sync_copy(edge_hbm.at[pl.ds(off, _CH)], idx_src)
            pltpu.sync_copy(edge_hbm.at[pl.ds(_E + off, _CH)], idx_dst)
            pltpu.async_copy(feat_hbm.at[idx_src], rows, sem).wait()
            pltpu.sync_copy(rows, acc_sh.at[idx_dst], add=True)
            return 0

        lax.fori_loop(0, _NCHUNK, body, 0)
        plsc.subcore_barrier()

        obase = pl.multiple_of(sid * _RPT, 8)
        pltpu.sync_copy(acc_sh.at[pl.ds(obase, _RPT)],
                        acc_out.at[cid, pl.ds(obase, _RPT)])

    return k(feat, edge_flat)


def _dense_tc(partials, counts, feat, Wl, Wr, b, relu):
    def body(p_ref, c_ref, f_ref, wl_ref, wr_ref, b_ref, o_ref):
        cnt = jnp.sum(c_ref[...], axis=0)
        recip = 1.0 / jnp.maximum(cnt, 1.0)
        agg = jnp.sum(p_ref[...], axis=0) * recip
        y = (jnp.dot(agg, wl_ref[...], preferred_element_type=jnp.float32)
             + jnp.dot(f_ref[...], wr_ref[...],
                       preferred_element_type=jnp.float32)
             + b_ref[...])
        if relu:
            y = jnp.maximum(y, 0.0)
        o_ref[...] = y

    return pl.pallas_call(
        body,
        grid=(_NP // _RB,),
        in_specs=[
            pl.BlockSpec((_NC, _RB, _D), lambda i: (0, i, 0)),
            pl.BlockSpec((_NC, _RB, 1), lambda i: (0, i, 0)),
            pl.BlockSpec((_RB, _D), lambda i: (i, 0)),
            pl.BlockSpec((_D, _D), lambda i: (0, 0)),
            pl.BlockSpec((_D, _D), lambda i: (0, 0)),
            pl.BlockSpec((1, _D), lambda i: (0, 0)),
        ],
        out_specs=pl.BlockSpec((_RB, _D), lambda i: (i, 0)),
        out_shape=jax.ShapeDtypeStruct((_N, _D), jnp.float32),
    )(partials, counts, feat, Wl, Wr, b)


def kernel(x, edge_index, Wl1, Wr1, b1, Wl2, Wr2, b2):
    eflat = edge_index.reshape(-1)
    cnt = _count_sc(eflat)[:, :, :1]
    p1 = _seg_sum_sc(x, eflat)
    h = _dense_tc(p1, cnt, x, Wl1, Wr1, b1.reshape(1, _D), relu=True)
    p2 = _seg_sum_sc(h, eflat)
    out = _dense_tc(p2, cnt, h, Wl2, Wr2, b2.reshape(1, _D), relu=False)
    return out


# speedup vs baseline: 8.8317x; 1.7931x over previous
import functools

import jax
import jax.numpy as jnp
from jax import lax
from jax.experimental import pallas as pl
from jax.experimental.pallas import tpu as pltpu
from jax.experimental.pallas import tpu_sc as plsc

_N = 10000
_E = 320000
_D = 128

_NC = 2
_NS = 16
_NW = _NC * _NS

_NP = 10240
_EPT = _E // _NW
_CH = 80
_NCHUNK = _EPT // _CH
_RPT = _NP // _NS
_RB = 1024


def _zero_fill(ref, val):
    v = jnp.full((16,), val, jnp.float32)

    def body(r, _):
        for j in range(_D // 16):
            ref[r, pl.ds(j * 16, 16)] = v
        return 0

    lax.fori_loop(0, _CH, body, 0)


def _count_sc(dst3d):
    mesh = plsc.VectorSubcoreMesh(
        core_axis_name="c", subcore_axis_name="s", num_cores=_NC)

    @functools.partial(
        pl.kernel,
        mesh=mesh,
        out_type=jax.ShapeDtypeStruct((_NC, _NP, _D), jnp.float32),
        scratch_types=[
            pltpu.VMEM((_NCHUNK, _CH), jnp.int32),
            pltpu.VMEM((_CH, _D), jnp.float32),
            pltpu.VMEM((_CH, _D), jnp.float32),
            pltpu.VMEM_SHARED((_NP, _D), jnp.float32),
            pltpu.SemaphoreType.DMA,
            pltpu.SemaphoreType.DMA,
        ],
    )
    def k(dst_hbm, cnt_out, dst_all, ones_b, zeros_b, cnt_sh, semA, semB):
        cid = lax.axis_index("c")
        sid = lax.axis_index("s")
        wid = cid * _NS + sid
        pltpu.sync_copy(dst_hbm.at[wid], dst_all)
        _zero_fill(ones_b, 1.0)
        _zero_fill(zeros_b, 0.0)
        for j in range(_RPT // _CH):
            pltpu.sync_copy(
                zeros_b, cnt_sh.at[pl.ds(sid * _RPT + j * _CH, _CH)])
        plsc.subcore_barrier()

        def fire(c, sem):
            pltpu.make_async_copy(
                ones_b, cnt_sh.at[dst_all.at[c]], sem).start(add=True)

        def drain(sem):
            pltpu.make_async_copy(
                ones_b, cnt_sh.at[dst_all.at[0]], sem).wait()

        fire(0, semA)
        fire(1, semB)

        def body(i, _):
            drain(semA)
            fire(2 + 2 * i, semA)
            drain(semB)
            fire(3 + 2 * i, semB)
            return 0

        npairs = (_NCHUNK - 2) // 2
        lax.fori_loop(0, npairs, body, 0)
        for c in range(2 + 2 * npairs, _NCHUNK):
            drain(semA)
            fire(c, semA)
        drain(semA)
        drain(semB)
        plsc.subcore_barrier()

        obase = pl.multiple_of(sid * _RPT, 8)
        pltpu.sync_copy(cnt_sh.at[pl.ds(obase, _RPT)],
                        cnt_out.at[cid, pl.ds(obase, _RPT)])

    return k(dst3d)


def _seg_sum_sc(feat, src2d, dst3d):
    mesh = plsc.VectorSubcoreMesh(
        core_axis_name="c", subcore_axis_name="s", num_cores=_NC)

    @functools.partial(
        pl.kernel,
        mesh=mesh,
        out_type=jax.ShapeDtypeStruct((_NC, _NP, _D), jnp.float32),
        scratch_types=[
            pltpu.VMEM((_EPT,), jnp.int32),
            pltpu.VMEM((_CH,), jnp.int32),
            pltpu.VMEM((_CH,), jnp.int32),
            pltpu.VMEM((_CH, _D), jnp.float32),
            pltpu.VMEM((_CH, _D), jnp.float32),
            pltpu.VMEM_SHARED((_NP, _D), jnp.float32),
            pltpu.SemaphoreType.DMA,
            pltpu.SemaphoreType.DMA,
            pltpu.SemaphoreType.DMA,
            pltpu.SemaphoreType.DMA,
            pltpu.SemaphoreType.DMA,
            pltpu.SemaphoreType.DMA,
        ],
    )
    def k(feat_hbm, src_hbm, dst_hbm, acc_out, src_all, idxA, idxB,
          rowsA, rowsB, acc_sh, gA, gB, sA, sB, iA, iB):
        cid = lax.axis_index("c")
        sid = lax.axis_index("s")
        wid = cid * _NS + sid
        pltpu.sync_copy(src_hbm.at[wid], src_all)
        _zero_fill(rowsA, 0.0)
        _zero_fill(rowsB, 0.0)
        for j in range(_RPT // _CH):
            pltpu.sync_copy(
                rowsA, acc_sh.at[pl.ds(sid * _RPT + j * _CH, _CH)])
        plsc.subcore_barrier()

        def gather(c, rows, sem):
            off = pl.multiple_of(c * _CH, 8)
            pltpu.make_async_copy(
                feat_hbm.at[src_all.at[pl.ds(off, _CH)]], rows, sem).start()

        def gwait(rows, sem):
            pltpu.make_async_copy(
                feat_hbm.at[src_all.at[pl.ds(0, _CH)]], rows, sem).wait()

        def idx_start(c, idx, sem):
            pltpu.make_async_copy(dst_hbm.at[wid, c], idx, sem).start()

        def idx_wait(idx, sem):
            pltpu.make_async_copy(dst_hbm.at[wid, 0], idx, sem).wait()

        def scat(rows, idx, sem):
            pltpu.make_async_copy(rows, acc_sh.at[idx], sem).start(add=True)

        def swait(rows, idx, sem):
            pltpu.make_async_copy(rows, acc_sh.at[idx], sem).wait()

        pltpu.sync_copy(dst_hbm.at[wid, 0], idxB)
        scat(rowsB, idxB, sB)
        idx_start(0, idxA, iA)

        def body(i, _):
            a = 2 * i
            gather(a, rowsA, gA)
            swait(rowsB, idxB, sB)
            idx_start(a + 1, idxB, iB)
            gwait(rowsA, gA)
            idx_wait(idxA, iA)
            scat(rowsA, idxA, sA)
            gather(a + 1, rowsB, gB)
            gwait(rowsB, gB)
            swait(rowsA, idxA, sA)
            nxt = jnp.minimum(a + 2, _NCHUNK - 1)
            idx_start(nxt, idxA, iA)
            idx_wait(idxB, iB)
            scat(rowsB, idxB, sB)
            return 0

        npairs = _NCHUNK // 2
        lax.fori_loop(0, npairs, body, 0)
        for c in range(2 * npairs, _NCHUNK):
            gather(c, rowsA, gA)
            swait(rowsB, idxB, sB)
            gwait(rowsA, gA)
            idx_wait(idxA, iA)
            scat(rowsA, idxA, sA)
            swait(rowsA, idxA, sA)
        if _NCHUNK % 2 == 0:
            swait(rowsB, idxB, sB)
            idx_wait(idxA, iA)
        plsc.subcore_barrier()

        obase = pl.multiple_of(sid * _RPT, 8)
        pltpu.sync_copy(acc_sh.at[pl.ds(obase, _RPT)],
                        acc_out.at[cid, pl.ds(obase, _RPT)])

    return k(feat, src2d, dst3d)


def _dense_tc(partials, counts, feat, Wl, Wr, b, relu):
    def body(p_ref, c_ref, f_ref, wl_ref, wr_ref, b_ref, o_ref):
        cnt = jnp.sum(c_ref[...], axis=0)
        recip = 1.0 / jnp.maximum(cnt, 1.0)
        agg = jnp.sum(p_ref[...], axis=0) * recip
        y = (jnp.dot(agg, wl_ref[...], preferred_element_type=jnp.float32)
             + jnp.dot(f_ref[...], wr_ref[...],
                       preferred_element_type=jnp.float32)
             + b_ref[...])
        if relu:
            y = jnp.maximum(y, 0.0)
        o_ref[...] = y

    return pl.pallas_call(
        body,
        grid=(_NP // _RB,),
        in_specs=[
            pl.BlockSpec((_NC, _RB, _D), lambda i: (0, i, 0)),
            pl.BlockSpec((_NC, _RB, 1), lambda i: (0, i, 0)),
            pl.BlockSpec((_RB, _D), lambda i: (i, 0)),
            pl.BlockSpec((_D, _D), lambda i: (0, 0)),
            pl.BlockSpec((_D, _D), lambda i: (0, 0)),
            pl.BlockSpec((1, _D), lambda i: (0, 0)),
        ],
        out_specs=pl.BlockSpec((_RB, _D), lambda i: (i, 0)),
        out_shape=jax.ShapeDtypeStruct((_N, _D), jnp.float32),
    )(partials, counts, feat, Wl, Wr, b)


def kernel(x, edge_index, Wl1, Wr1, b1, Wl2, Wr2, b2):
    src2d = edge_index[0].reshape(_NW, _EPT)
    dst3d = edge_index[1].reshape(_NW, _NCHUNK, _CH)
    cnt = _count_sc(dst3d)[:, :, :1]
    p1 = _seg_sum_sc(x, src2d, dst3d)
    h = _dense_tc(p1, cnt, x, Wl1, Wr1, b1.reshape(1, _D), relu=True)
    p2 = _seg_sum_sc(h, src2d, dst3d)
    out = _dense_tc(p2, cnt, h, Wl2, Wr2, b2.reshape(1, _D), relu=False)
    return out
